# baseline (speedup 1.0000x reference)
import jax
import jax.numpy as jnp
from jax import lax
from jax.experimental import pallas as pl
from jax.experimental.pallas import tpu as pltpu
from jax.experimental.pallas import tpu_sc as plsc

_L = 16


def _pos_body(col_hbm, row_hbm, out_hbm, col_v, row_v, content_v, sem):
    n_ch = content_v.shape[0]
    hw = content_v.shape[1]
    w = 32
    batches = out_hbm.shape[0] // (32 * n_ch)

    pltpu.sync_copy(col_hbm, col_v)
    pltpu.sync_copy(row_hbm, row_v)

    wid = lax.axis_index("s") * 2 + lax.axis_index("c")
    is_x = wid < 16
    c0 = lax.rem(wid, 16) * n_ch

    iota = lax.iota(jnp.int32, _L)
    zeros = jnp.zeros((_L,), jnp.int32)
    cidx = [zeros + (c0 + ci) for ci in range(n_ch)]

    @pl.when(is_x)
    def _():
        for ci in range(n_ch):
            v_lo = plsc.load_gather(col_v, [iota, cidx[ci]])
            v_hi = plsc.load_gather(col_v, [iota + _L, cidx[ci]])

            def fill(k, carry, ci=ci, v_lo=v_lo, v_hi=v_hi):
                content_v[ci, pl.ds(k * 2 * _L, _L)] = v_lo
                content_v[ci, pl.ds(k * 2 * _L + _L, _L)] = v_hi
                return carry

            lax.fori_loop(0, hw // (2 * _L), fill, 0)

    @pl.when(jnp.logical_not(is_x))
    def _():
        def fill(y, carry):
            yidx = zeros + y
            for ci in range(n_ch):
                v = plsc.load_gather(row_v, [yidx, cidx[ci]])
                content_v[ci, pl.ds(y * 2 * _L, _L)] = v
                content_v[ci, pl.ds(y * 2 * _L + _L, _L)] = v
            return carry

        lax.fori_loop(0, hw // w, fill, 0)

    copies = []
    for b in range(batches):
        dst = out_hbm.at[pl.ds(b * (32 * n_ch) + wid * n_ch, n_ch), :]
        copies.append(pltpu.async_copy(content_v, dst, sem))
    for cp in copies:
        cp.wait()


@jax.jit
def kernel(pixel_values, row_embeddings, column_embeddings):
    b = pixel_values.shape[0]
    h, w = pixel_values.shape[-2], pixel_values.shape[-1]
    d = column_embeddings.shape[-1]

    run = pl.kernel(
        _pos_body,
        out_type=jax.ShapeDtypeStruct((b * 2 * d, h * w), jnp.float32),
        mesh=plsc.VectorSubcoreMesh(core_axis_name="c", subcore_axis_name="s"),
        compiler_params=pltpu.CompilerParams(
            use_tc_tiling_on_sc=False, needs_layout_passes=False
        ),
        scratch_types=[
            pltpu.VMEM(column_embeddings.shape, jnp.float32),
            pltpu.VMEM(row_embeddings.shape, jnp.float32),
            pltpu.VMEM((16, 1024), jnp.float32),
            pltpu.SemaphoreType.DMA,
        ],
    )
    out = run(column_embeddings, row_embeddings)
    return out.reshape(b, 2 * d, h, w)
